# baseline (speedup 1.0000x reference)
import functools
import math

import jax
import jax.numpy as jnp
from jax import lax
from jax.experimental import pallas as pl
from jax.experimental.pallas import tpu as pltpu
from jax.experimental.pallas import tpu_sc as plsc

NC = 2
NS = 16
NW = NC * NS
CH = 128
LANES = 16


@functools.partial(jax.jit, static_argnums=(2, 3, 4))
def _emb_lookup(x3, table, b_per_w, n_chunks, d_model):
    scale = math.sqrt(d_model)
    mesh = plsc.VectorSubcoreMesh(core_axis_name="c", subcore_axis_name="s")

    @functools.partial(
        pl.kernel,
        mesh=mesh,
        out_type=jax.ShapeDtypeStruct((NW * b_per_w, d_model), jnp.float32),
        scratch_types=[
            pltpu.VMEM((n_chunks, CH), jnp.int32),
            pltpu.VMEM((CH, d_model), jnp.float32),
            pltpu.SemaphoreType.DMA,
        ],
        compiler_params=pltpu.CompilerParams(use_tc_tiling_on_sc=False),
    )
    def emb_kernel(x_hbm, table_hbm, out_hbm, idx_v, rows_v, sem):
        wid = lax.axis_index("s") * NC + lax.axis_index("c")
        base = wid * b_per_w
        pltpu.sync_copy(x_hbm.at[wid], idx_v)

        def chunk_body(c, carry):
            pltpu.async_copy(table_hbm.at[idx_v.at[c]], rows_v, sem).wait()

            def scale_row(i, carry2):
                for j in range(d_model // LANES):
                    sl = pl.ds(j * LANES, LANES)
                    rows_v[i, sl] = rows_v[i, sl] * scale
                return carry2

            lax.fori_loop(0, CH, scale_row, 0, unroll=2)
            pltpu.sync_copy(rows_v, out_hbm.at[pl.ds(base + c * CH, CH)])
            return carry

        lax.fori_loop(0, n_chunks, chunk_body, 0)

    return emb_kernel(x3, table)


def kernel(x, table):
    bsz, seq = x.shape
    vocab, d_model = table.shape
    B = bsz * seq
    assert B % (NW * CH) == 0
    b_per_w = B // NW
    n_chunks = b_per_w // CH
    x3 = x.reshape(NW, n_chunks, CH).astype(jnp.int32)
    out = _emb_lookup(x3, table, b_per_w, n_chunks, d_model)
    return out.reshape(bsz, seq, d_model)


# speedup vs baseline: 1.1584x; 1.1584x over previous
import functools
import math

import jax
import jax.numpy as jnp
from jax import lax
from jax.experimental import pallas as pl
from jax.experimental.pallas import tpu as pltpu
from jax.experimental.pallas import tpu_sc as plsc

NC = 2
NS = 16
NW = NC * NS
CH = 128
NBUF = 8
LANES = 16


@functools.partial(jax.jit, static_argnums=(2, 3, 4))
def _emb_lookup(x3, table, b_per_w, n_chunks, d_model):
    scale = math.sqrt(d_model)
    n_outer = n_chunks // NBUF
    mesh = plsc.VectorSubcoreMesh(core_axis_name="c", subcore_axis_name="s")

    @functools.partial(
        pl.kernel,
        mesh=mesh,
        out_type=jax.ShapeDtypeStruct((NW * b_per_w, d_model), jnp.float32),
        scratch_types=[
            pltpu.VMEM((n_chunks, CH), jnp.int32),
            pltpu.VMEM((NBUF, CH, d_model), jnp.float32),
            [pltpu.SemaphoreType.DMA] * NBUF,
            [pltpu.SemaphoreType.DMA] * NBUF,
        ],
        compiler_params=pltpu.CompilerParams(use_tc_tiling_on_sc=False),
    )
    def emb_kernel(x_hbm, table_hbm, out_hbm, idx_v, rows_v, gsems, osems):
        wid = lax.axis_index("s") * NC + lax.axis_index("c")
        base = wid * b_per_w
        pltpu.sync_copy(x_hbm.at[wid], idx_v)

        def gather(c, b):
            return pltpu.make_async_copy(
                table_hbm.at[idx_v.at[c]], rows_v.at[b], gsems[b]
            )

        def writeback(c, b):
            return pltpu.make_async_copy(
                rows_v.at[b], out_hbm.at[pl.ds(base + c * CH, CH)], osems[b]
            )

        def outer(co, carry):
            c0 = co * NBUF
            for b in range(NBUF):
                @pl.when(co > 0)
                def _drain():
                    writeback(c0 - NBUF + b, b).wait()

                gather(c0 + b, b).start()

            for b in range(NBUF):
                gather(c0 + b, b).wait()

                def scale_row(i, carry2):
                    for j in range(d_model // LANES):
                        sl = pl.ds(j * LANES, LANES)
                        rows_v[b, i, sl] = rows_v[b, i, sl] * scale
                    return carry2

                lax.fori_loop(0, CH, scale_row, 0, unroll=2)
                writeback(c0 + b, b).start()
            return carry

        lax.fori_loop(0, n_outer, outer, 0)
        for b in range(NBUF):
            writeback((n_outer - 1) * NBUF + b, b).wait()

    return emb_kernel(x3, table)


def kernel(x, table):
    bsz, seq = x.shape
    vocab, d_model = table.shape
    B = bsz * seq
    assert B % (NW * CH * NBUF) == 0
    b_per_w = B // NW
    n_chunks = b_per_w // CH
    x3 = x.reshape(NW, n_chunks, CH).astype(jnp.int32)
    out = _emb_lookup(x3, table, b_per_w, n_chunks, d_model)
    return out.reshape(bsz, seq, d_model)
